# baseline (speedup 1.0000x reference)
import functools

import jax
import jax.numpy as jnp
from jax.experimental import pallas as pl
from jax.experimental.pallas import tpu as pltpu

E = 16
K = 2
T = 128
D = 1024
F = 2048
FH = F // 2


def _silu(g):
    return g * jax.nn.sigmoid(g)


def _moe_kernel(route_ref, rw_ref, x_ref,
                w1a_ref, w1b_ref, w3a_ref, w3b_ref,
                w2a_ref, w2b_ref, out_ref):
    e = pl.program_id(0)

    @pl.when(e == 0)
    def _():
        out_ref[:, :] = jnp.zeros_like(out_ref)

    x = x_ref[:, :].astype(jnp.bfloat16)

    def mm(a, b_ref):
        return jax.lax.dot_general(
            a, b_ref[0, 0].astype(jnp.bfloat16),
            (((1,), (1,)), ((), ())),
            preferred_element_type=jnp.float32)

    g0 = mm(x, w1a_ref)
    u0 = mm(x, w3a_ref)
    h0 = (_silu(g0) * u0).astype(jnp.bfloat16)
    y = mm(h0, w2a_ref)
    g1 = mm(x, w1b_ref)
    u1 = mm(x, w3b_ref)
    h1 = (_silu(g1) * u1).astype(jnp.bfloat16)
    y += mm(h1, w2b_ref)

    sel = (route_ref[:, :] == e).astype(jnp.float32)
    combine = jnp.sum(sel * rw_ref[:, :], axis=1, keepdims=True)

    out_ref[:, :] += combine * y


@jax.jit
def kernel(hidden_states, expert_routing_table, router_weights, w13, w2):
    route = expert_routing_table.astype(jnp.int32)
    w13q = w13.reshape(E, 4, FH, D)
    w2r = w2.reshape(E, 1, D, F)

    wq_spec = lambda q: pl.BlockSpec((1, 1, FH, D), lambda e, q=q: (e, q, 0, 0))
    w2_spec = lambda h: pl.BlockSpec((1, 1, D, FH), lambda e, h=h: (e, 0, 0, h))

    out = pl.pallas_call(
        _moe_kernel,
        grid=(E,),
        in_specs=[
            pl.BlockSpec((T, K), lambda e: (0, 0)),
            pl.BlockSpec((T, K), lambda e: (0, 0)),
            pl.BlockSpec((T, D), lambda e: (0, 0)),
            wq_spec(0),
            wq_spec(1),
            wq_spec(2),
            wq_spec(3),
            w2_spec(0),
            w2_spec(1),
        ],
        out_specs=pl.BlockSpec((T, D), lambda e: (0, 0)),
        out_shape=jax.ShapeDtypeStruct((T, D), jnp.float32),
        compiler_params=pltpu.CompilerParams(
            dimension_semantics=("arbitrary",),
        ),
    )(route, router_weights, hidden_states,
      w13q, w13q, w13q, w13q, w2r, w2r)
    return out


# speedup vs baseline: 1.0107x; 1.0107x over previous
import functools

import jax
import jax.numpy as jnp
from jax.experimental import pallas as pl
from jax.experimental.pallas import tpu as pltpu

E = 16
K = 2
T = 128
D = 1024
F = 2048


def _silu(g):
    return g * jax.nn.sigmoid(g)


def _moe_kernel(route_ref, rw_ref, x_ref, w1_ref, w3_ref, w2_ref, out_ref):
    e = pl.program_id(0)

    @pl.when(e == 0)
    def _():
        out_ref[:, :] = jnp.zeros_like(out_ref)

    x = x_ref[:, :].astype(jnp.bfloat16)

    def mm(a, b):
        return jax.lax.dot_general(
            a, b.astype(jnp.bfloat16),
            (((1,), (1,)), ((), ())),
            preferred_element_type=jnp.float32)

    g = mm(x, w1_ref[0, 0])
    u = mm(x, w3_ref[0, 0])
    h = (_silu(g) * u).astype(jnp.bfloat16)
    y = mm(h, w2_ref[0])

    sel = (route_ref[:, :] == e).astype(jnp.float32)
    combine = jnp.sum(sel * rw_ref[:, :], axis=1, keepdims=True)

    out_ref[:, :] += combine * y


@jax.jit
def kernel(hidden_states, expert_routing_table, router_weights, w13, w2):
    route = expert_routing_table.astype(jnp.int32)
    w13r = w13.reshape(E, 2, F, D)

    out = pl.pallas_call(
        _moe_kernel,
        grid=(E,),
        in_specs=[
            pl.BlockSpec((T, K), lambda e: (0, 0)),
            pl.BlockSpec((T, K), lambda e: (0, 0)),
            pl.BlockSpec((T, D), lambda e: (0, 0)),
            pl.BlockSpec((1, 1, F, D), lambda e: (e, 0, 0, 0)),
            pl.BlockSpec((1, 1, F, D), lambda e: (e, 1, 0, 0)),
            pl.BlockSpec((1, D, F), lambda e: (e, 0, 0)),
        ],
        out_specs=pl.BlockSpec((T, D), lambda e: (0, 0)),
        out_shape=jax.ShapeDtypeStruct((T, D), jnp.float32),
        compiler_params=pltpu.CompilerParams(
            dimension_semantics=("arbitrary",),
        ),
    )(route, router_weights, hidden_states, w13r, w13r, w2)
    return out
